# baseline (device time: 67284 ns/iter reference)
import jax
import jax.numpy as jnp
from jax import lax
from jax.experimental import pallas as pl
from jax.experimental.pallas import tpu as pltpu

N_DEV = 8
BLK = 64
NEG_INF = -1e9


def kernel(x, Wq, K_ext, V_ext, Wo):
    B, Sq, D = x.shape
    _, Skv, HL, Dh = K_ext.shape
    DL = HL * Dh

    def body(x_ref, wq_ref, k_ref, v_ref, wo_ref, out_ref,
             comm_ref, send_sems, recv_sems):
        my = lax.axis_index("i")
        left = lax.rem(my + N_DEV - 1, N_DEV)
        right = lax.rem(my + 1, N_DEV)

        barrier = pltpu.get_barrier_semaphore()
        for nbr in (left, right):
            pl.semaphore_signal(barrier, inc=1, device_id=(nbr,),
                                device_id_type=pl.DeviceIdType.MESH)
        pl.semaphore_wait(barrier, 2)

        xs = x_ref[:].reshape(B * Sq, D)
        wq = wq_ref[:, pl.ds(my * DL, DL)]
        q = jnp.dot(xs, wq, preferred_element_type=jnp.float32)
        q = q.reshape(B, Sq, HL, Dh)
        k = k_ref[:]
        v = v_ref[:]

        qb = lax.broadcasted_iota(jnp.int32, (Sq, Skv), 0) // BLK
        kb = lax.broadcasted_iota(jnp.int32, (Sq, Skv), 1) // BLK
        mask = kb <= qb

        ctx_rows = []
        for b in range(B):
            heads = []
            for h in range(HL):
                qh = q[b, :, h, :]
                kh = k[b, :, h, :]
                vh = v[b, :, h, :]
                s = lax.dot_general(
                    qh, kh, (((1,), (1,)), ((), ())),
                    preferred_element_type=jnp.float32) * 0.125
                s = jnp.where(mask, s, NEG_INF)
                m = jnp.max(s, axis=-1, keepdims=True)
                w = jnp.exp(s - m)
                w = w / jnp.sum(w, axis=-1, keepdims=True)
                heads.append(lax.dot_general(
                    w, vh, (((1,), (0,)), ((), ())),
                    preferred_element_type=jnp.float32))
            ctx_rows.append(jnp.concatenate(heads, axis=1))
        ctx = jnp.concatenate(ctx_rows, axis=0)

        wo = wo_ref[pl.ds(my * DL, DL), :]
        partial = jnp.dot(ctx, wo, preferred_element_type=jnp.float32)
        partial = partial.reshape(B, Sq, D)

        out_ref[:] = partial
        comm_ref[0] = partial

        for h in range(N_DEV - 1):
            rdma = pltpu.make_async_remote_copy(
                src_ref=comm_ref.at[h],
                dst_ref=comm_ref.at[h + 1],
                send_sem=send_sems.at[h],
                recv_sem=recv_sems.at[h],
                device_id=(right,),
                device_id_type=pl.DeviceIdType.MESH,
            )
            rdma.start()
            rdma.wait()
            out_ref[:] += comm_ref[h + 1]

    return pl.pallas_call(
        body,
        out_shape=jax.ShapeDtypeStruct((B, Sq, D), jnp.float32),
        in_specs=[pl.BlockSpec(memory_space=pltpu.VMEM)] * 5,
        out_specs=pl.BlockSpec(memory_space=pltpu.VMEM),
        scratch_shapes=[
            pltpu.VMEM((N_DEV, B, Sq, D), jnp.float32),
            pltpu.SemaphoreType.DMA((N_DEV - 1,)),
            pltpu.SemaphoreType.DMA((N_DEV - 1,)),
        ],
        compiler_params=pltpu.CompilerParams(collective_id=0),
    )(x, Wq, K_ext, V_ext, Wo)


# device time: 26735 ns/iter; 2.5167x vs baseline; 2.5167x over previous
import jax
import jax.numpy as jnp
from jax import lax
from jax.experimental import pallas as pl
from jax.experimental.pallas import tpu as pltpu

N_DEV = 8
BLK = 64
NEG_INF = -1e9


def kernel(x, Wq, K_ext, V_ext, Wo):
    B, Sq, D = x.shape
    _, Skv, HL, Dh = K_ext.shape
    DL = HL * Dh
    M = B * Sq
    R = M // N_DEV

    def body(x_ref, wq_ref, k_ref, v_ref, wo_ref, out_ref,
             part_ref, comm_ref, acc_ref,
             p1_send, p1_recv, p2_send, p2_recv):
        my = lax.axis_index("i")

        barrier = pltpu.get_barrier_semaphore()
        for o in range(1, N_DEV):
            tgt = lax.rem(my + o, N_DEV)
            pl.semaphore_signal(barrier, inc=1, device_id=(tgt,),
                                device_id_type=pl.DeviceIdType.MESH)
        pl.semaphore_wait(barrier, N_DEV - 1)

        xs = x_ref[:].reshape(M, D)
        wq = wq_ref[:, pl.ds(my * DL, DL)]
        q = jnp.dot(xs, wq, preferred_element_type=jnp.float32)
        q = q.reshape(B, Sq, HL, Dh)
        k = k_ref[:]
        v = v_ref[:]

        qb = lax.broadcasted_iota(jnp.int32, (Sq, Skv), 0) // BLK
        kb = lax.broadcasted_iota(jnp.int32, (Sq, Skv), 1) // BLK
        mask = kb <= qb

        ctx_rows = []
        for b in range(B):
            heads = []
            for h in range(HL):
                qh = q[b, :, h, :]
                kh = k[b, :, h, :]
                vh = v[b, :, h, :]
                s = lax.dot_general(
                    qh, kh, (((1,), (1,)), ((), ())),
                    preferred_element_type=jnp.float32) * 0.125
                s = jnp.where(mask, s, NEG_INF)
                m = jnp.max(s, axis=-1, keepdims=True)
                w = jnp.exp(s - m)
                w = w / jnp.sum(w, axis=-1, keepdims=True)
                heads.append(lax.dot_general(
                    w, vh, (((1,), (0,)), ((), ())),
                    preferred_element_type=jnp.float32))
            ctx_rows.append(jnp.concatenate(heads, axis=1))
        ctx = jnp.concatenate(ctx_rows, axis=0)

        wo = wo_ref[pl.ds(my * DL, DL), :]
        part_ref[:] = jnp.dot(ctx, wo, preferred_element_type=jnp.float32)

        sends = []
        for o in range(1, N_DEV):
            tgt = lax.rem(my + o, N_DEV)
            rd = pltpu.make_async_remote_copy(
                src_ref=part_ref.at[pl.ds(tgt * R, R), :],
                dst_ref=comm_ref.at[o - 1],
                send_sem=p1_send.at[o - 1],
                recv_sem=p1_recv.at[o - 1],
                device_id=(tgt,),
                device_id_type=pl.DeviceIdType.MESH,
            )
            rd.start()
            sends.append(rd)

        for o in range(1, N_DEV):
            pltpu.make_async_remote_copy(
                src_ref=part_ref.at[pl.ds(0, R), :],
                dst_ref=comm_ref.at[o - 1],
                send_sem=p1_send.at[o - 1],
                recv_sem=p1_recv.at[o - 1],
                device_id=(my,),
                device_id_type=pl.DeviceIdType.MESH,
            ).wait_recv()

        red = part_ref[pl.ds(my * R, R), :]
        for o in range(1, N_DEV):
            red = red + comm_ref[o - 1]
        acc_ref[pl.ds(my * R, R), :] = red

        for o in range(1, N_DEV):
            tgt = lax.rem(my + o, N_DEV)
            rd = pltpu.make_async_remote_copy(
                src_ref=acc_ref.at[pl.ds(my * R, R), :],
                dst_ref=acc_ref.at[pl.ds(my * R, R), :],
                send_sem=p2_send.at[o - 1],
                recv_sem=p2_recv.at[o - 1],
                device_id=(tgt,),
                device_id_type=pl.DeviceIdType.MESH,
            )
            rd.start()
            sends.append(rd)

        for o in range(1, N_DEV):
            src_dev = lax.rem(my - o + N_DEV, N_DEV)
            pltpu.make_async_remote_copy(
                src_ref=acc_ref.at[pl.ds(0, R), :],
                dst_ref=acc_ref.at[pl.ds(src_dev * R, R), :],
                send_sem=p2_send.at[o - 1],
                recv_sem=p2_recv.at[o - 1],
                device_id=(my,),
                device_id_type=pl.DeviceIdType.MESH,
            ).wait_recv()

        for rd in sends:
            rd.wait_send()
        out_ref[:] = acc_ref[:].reshape(B, Sq, D)

    return pl.pallas_call(
        body,
        out_shape=jax.ShapeDtypeStruct((B, Sq, D), jnp.float32),
        in_specs=[pl.BlockSpec(memory_space=pltpu.VMEM)] * 5,
        out_specs=pl.BlockSpec(memory_space=pltpu.VMEM),
        scratch_shapes=[
            pltpu.VMEM((M, D), jnp.float32),
            pltpu.VMEM((N_DEV - 1, R, D), jnp.float32),
            pltpu.VMEM((M, D), jnp.float32),
            pltpu.SemaphoreType.DMA((N_DEV - 1,)),
            pltpu.SemaphoreType.DMA((N_DEV - 1,)),
            pltpu.SemaphoreType.DMA((N_DEV - 1,)),
            pltpu.SemaphoreType.DMA((N_DEV - 1,)),
        ],
        compiler_params=pltpu.CompilerParams(collective_id=0),
    )(x, Wq, K_ext, V_ext, Wo)


# device time: 21498 ns/iter; 3.1298x vs baseline; 1.2436x over previous
import jax
import jax.numpy as jnp
from jax import lax
from jax.experimental import pallas as pl
from jax.experimental.pallas import tpu as pltpu

N_DEV = 8
BLK = 64
NEG_INF = -1e9
OFFSETS = (6, 2, 5, 7, 1, 3, 4)


def kernel(x, Wq, K_ext, V_ext, Wo):
    B, Sq, D = x.shape
    _, Skv, HL, Dh = K_ext.shape
    DL = HL * Dh
    DF = N_DEV * DL
    M = B * Sq
    R = M // N_DEV

    def body(x_ref, wq_ref, k_ref, v_ref, wo_ref, out_ref,
             wq_v, wo_v, wo_bf, x_v, out_v, ctx_ref, comm_ref, red_ref,
             acc_ref, w_sems, p1_send, p1_recv, p2_send, p2_recv):
        my = lax.axis_index("i")

        x_cp = pltpu.make_async_copy(x_ref, x_v, w_sems.at[2])
        x_cp.start()
        wq_cp = pltpu.make_async_copy(
            wq_ref.at[:, pl.ds(my * DL, DL)], wq_v, w_sems.at[0])
        wo_cp = pltpu.make_async_copy(wo_ref, wo_v, w_sems.at[1])
        wq_cp.start()
        wo_cp.start()

        barrier = pltpu.get_barrier_semaphore()
        for o in OFFSETS:
            tgt = lax.rem(my + o, N_DEV)
            pl.semaphore_signal(barrier, inc=1, device_id=(tgt,),
                                device_id_type=pl.DeviceIdType.MESH)
        x_cp.wait()
        wq_cp.wait()

        xs = x_v[:].reshape(M, D).astype(jnp.bfloat16)
        wq = wq_v[:, :].astype(jnp.bfloat16)
        q = jnp.dot(xs, wq, preferred_element_type=jnp.float32)
        q = q.astype(jnp.bfloat16)
        k = k_ref[:].astype(jnp.bfloat16)
        v = v_ref[:].astype(jnp.bfloat16)

        ri = lax.broadcasted_iota(jnp.int32, (M, M), 0)
        ci = lax.broadcasted_iota(jnp.int32, (M, M), 1)
        mask = ((ri // Sq) == (ci // Sq)) & (
            ((ci % Sq) // BLK) <= ((ri % Sq) // BLK))

        heads = []
        for h in range(HL):
            qh = q[:, h * Dh:(h + 1) * Dh]
            kh = k[:, :, h, :].reshape(M, Dh)
            vh = v[:, :, h, :].reshape(M, Dh)
            s = lax.dot_general(
                qh, kh, (((1,), (1,)), ((), ())),
                preferred_element_type=jnp.float32) * 0.125
            s = jnp.where(mask, s, NEG_INF)
            m = jnp.max(s, axis=-1, keepdims=True)
            w = jnp.exp(s - m)
            w = (w / jnp.sum(w, axis=-1, keepdims=True)).astype(jnp.bfloat16)
            heads.append(lax.dot_general(
                w, vh, (((1,), (0,)), ((), ())),
                preferred_element_type=jnp.float32))
        ctx = jnp.concatenate(heads, axis=1).astype(jnp.bfloat16)
        ctx_ref[:] = ctx
        comm_ref[my] = ctx_ref[pl.ds(my * R, R), :]

        pl.semaphore_wait(barrier, N_DEV - 1)

        sends = []
        for o in OFFSETS:
            tgt = lax.rem(my + o, N_DEV)
            rd = pltpu.make_async_remote_copy(
                src_ref=ctx_ref.at[pl.ds(tgt * R, R), :],
                dst_ref=comm_ref.at[my],
                send_sem=p1_send.at[o - 1],
                recv_sem=p1_recv.at[my],
                device_id=(tgt,),
                device_id_type=pl.DeviceIdType.MESH,
            )
            rd.start()
            sends.append(rd)

        wo_cp.wait()
        wo_bf[:] = wo_v[:].astype(jnp.bfloat16)

        for o in OFFSETS:
            src_dev = lax.rem(my - o + N_DEV, N_DEV)
            pltpu.make_async_remote_copy(
                src_ref=ctx_ref.at[pl.ds(0, R), :],
                dst_ref=comm_ref.at[src_dev],
                send_sem=p1_send.at[o - 1],
                recv_sem=p1_recv.at[src_dev],
                device_id=(my,),
                device_id_type=pl.DeviceIdType.MESH,
            ).wait_recv()

        g = jnp.concatenate([comm_ref[j] for j in range(N_DEV)], axis=1)
        red = jnp.dot(g, wo_bf[:, :], preferred_element_type=jnp.float32)
        red_ref[:] = red.astype(jnp.bfloat16)

        for o in OFFSETS:
            tgt = lax.rem(my + o, N_DEV)
            rd = pltpu.make_async_remote_copy(
                src_ref=red_ref,
                dst_ref=acc_ref.at[pl.ds(my * R, R), :],
                send_sem=p2_send.at[o - 1],
                recv_sem=p2_recv.at[o - 1],
                device_id=(tgt,),
                device_id_type=pl.DeviceIdType.MESH,
            )
            rd.start()
            sends.append(rd)
        acc_ref[pl.ds(my * R, R), :] = red_ref[:]

        for o in OFFSETS:
            src_dev = lax.rem(my - o + N_DEV, N_DEV)
            pltpu.make_async_remote_copy(
                src_ref=red_ref,
                dst_ref=acc_ref.at[pl.ds(src_dev * R, R), :],
                send_sem=p2_send.at[o - 1],
                recv_sem=p2_recv.at[o - 1],
                device_id=(my,),
                device_id_type=pl.DeviceIdType.MESH,
            ).wait_recv()

        out_v[:] = acc_ref[:].astype(jnp.float32).reshape(B, Sq, D)
        out_cp = pltpu.make_async_copy(out_v, out_ref, w_sems.at[2])
        out_cp.start()
        for rd in sends:
            rd.wait_send()
        out_cp.wait()

    return pl.pallas_call(
        body,
        out_shape=jax.ShapeDtypeStruct((B, Sq, D), jnp.float32),
        in_specs=[
            pl.BlockSpec(memory_space=pltpu.MemorySpace.HBM),
            pl.BlockSpec(memory_space=pltpu.MemorySpace.HBM),
            pl.BlockSpec(memory_space=pltpu.VMEM),
            pl.BlockSpec(memory_space=pltpu.VMEM),
            pl.BlockSpec(memory_space=pltpu.MemorySpace.HBM),
        ],
        out_specs=pl.BlockSpec(memory_space=pltpu.MemorySpace.HBM),
        scratch_shapes=[
            pltpu.VMEM((D, DL), jnp.float32),
            pltpu.VMEM((DF, D), jnp.float32),
            pltpu.VMEM((DF, D), jnp.bfloat16),
            pltpu.VMEM((B, Sq, D), jnp.float32),
            pltpu.VMEM((B, Sq, D), jnp.float32),
            pltpu.VMEM((M, DL), jnp.bfloat16),
            pltpu.VMEM((N_DEV, R, DL), jnp.bfloat16),
            pltpu.VMEM((R, D), jnp.bfloat16),
            pltpu.VMEM((M, D), jnp.bfloat16),
            pltpu.SemaphoreType.DMA((3,)),
            pltpu.SemaphoreType.DMA((N_DEV - 1,)),
            pltpu.SemaphoreType.DMA((N_DEV,)),
            pltpu.SemaphoreType.DMA((N_DEV - 1,)),
            pltpu.SemaphoreType.DMA((N_DEV - 1,)),
        ],
        compiler_params=pltpu.CompilerParams(collective_id=0),
    )(x, Wq, K_ext, V_ext, Wo)
